# baseline (speedup 1.0000x reference)
import functools

import jax
import jax.numpy as jnp
from jax import lax
from jax.experimental import pallas as pl
from jax.experimental.pallas import tpu as pltpu
from jax.experimental.pallas import tpu_sc as plsc

_N = 10000
_E = 320000
_D = 128

_NW = 32
_ROWS = 313
_NPAD = _NW * _ROWS
_CHUNK = 1024
_SUBB = 128
_EPT = _E // _NW
_CAPR = 80
_BSTRIDE = _CHUNK + 16
_NSEG = _NW * _NW

_mesh = plsc.VectorSubcoreMesh(core_axis_name="c", subcore_axis_name="s")


def _wid():
    return lax.axis_index("s") * 2 + lax.axis_index("c")


def _splat(x):
    return jnp.full((16,), 1, jnp.int32) * x


@functools.partial(
    pl.kernel,
    out_type=[
        jax.ShapeDtypeStruct((_NSEG * _CAPR, 128), jnp.int32),
        jax.ShapeDtypeStruct((_NSEG * 16,), jnp.int32),
    ],
    mesh=_mesh,
    scratch_types=[
        pltpu.VMEM((_EPT + 16,), jnp.int32),
        pltpu.VMEM((_EPT + 16,), jnp.int32),
        pltpu.VMEM((_NW * _BSTRIDE,), jnp.int32),
        pltpu.VMEM((8, 128), jnp.int32),
        pltpu.VMEM((_NW * 16,), jnp.int32),
        pltpu.VMEM((_NW * 16,), jnp.int32),
        pltpu.VMEM((16,), jnp.int32),
    ],
)
def _bucketize(edges_hbm, bpk_hbm, cnt_hbm, sstage, dstage, pbuf, fbuf, fill,
               done, cbuf):
    t = _wid()
    zero16 = jnp.zeros((16,), jnp.int32)

    def zinit(b, _):
        fill[pl.ds(pl.multiple_of(b * 16, 16), 16)] = zero16
        done[pl.ds(pl.multiple_of(b * 16, 16), 16)] = zero16
        return 0

    lax.fori_loop(0, _NW, zinit, 0)

    eo = pl.multiple_of(t * _EPT, 16)
    pltpu.sync_copy(edges_hbm.at[pl.ds(eo, _EPT)], sstage.at[pl.ds(0, _EPT)])
    pltpu.sync_copy(edges_hbm.at[pl.ds(_E + eo, _EPT)],
                    dstage.at[pl.ds(0, _EPT)])

    def _flush_block(b, w0r):
        def mv(r8, _):
            for j in range(8):
                fbuf[r8, pl.ds(16 * j, 16)] = (
                    pbuf[pl.ds(b * _BSTRIDE + r8 * 128 + 16 * j, 16)])
            return 0

        lax.fori_loop(0, 8, mv, 0)
        seg = b * _NW + t
        ro = pl.multiple_of(seg * _CAPR + w0r, 8)
        pltpu.sync_copy(fbuf, bpk_hbm.at[pl.ds(ro, 8)])

    def edge(e, _):
        s = sstage[pl.ds(e, 16)][0]
        d = dstage[pl.ds(e, 16)][0]
        b = d // _ROWS
        loc = d - b * _ROWS
        c = fill[pl.ds(b * 16, 16)][0]
        pbuf[pl.ds(b * _BSTRIDE + c, 16)] = _splat(s * 512 + loc)

        def flush(_):
            w0 = done[pl.ds(b * 16, 16)][0]
            _flush_block(b, w0)
            done[pl.ds(b * 16, 16)] = _splat(w0 + 8)
            fill[pl.ds(b * 16, 16)] = zero16
            return 0

        def keep(_):
            fill[pl.ds(b * 16, 16)] = _splat(c + 1)
            return 0

        lax.cond(c + 1 >= _CHUNK, flush, keep, 0)
        return 0

    lax.fori_loop(0, _EPT, edge, 0)

    pad_d = jnp.full((16,), _ROWS, jnp.int32)

    def fin(b, _):
        c = fill[pl.ds(b * 16, 16)][0]
        for j in range(64):
            @pl.when(c + 16 * j < _CHUNK)
            def _():
                pbuf[pl.ds(b * _BSTRIDE + c + 16 * j, 16)] = pad_d
        w0 = done[pl.ds(b * 16, 16)][0]
        _flush_block(b, w0)
        cbuf[pl.ds(0, 16)] = _splat(w0 * 128 + c)
        co = pl.multiple_of((b * _NW + t) * 16, 16)
        pltpu.sync_copy(cbuf, cnt_hbm.at[pl.ds(co, 16)])
        return 0

    lax.fori_loop(0, _NW, fin, 0)


@functools.partial(
    pl.kernel,
    out_type=[
        jax.ShapeDtypeStruct((_NPAD * _D,), jnp.float32),
        jax.ShapeDtypeStruct((_NPAD * 16,), jnp.float32),
    ],
    mesh=_mesh,
    scratch_types=[
        pltpu.VMEM(((_ROWS + 1) * _D,), jnp.float32),
        pltpu.VMEM(((_ROWS + 1) * 16,), jnp.float32),
        pltpu.VMEM((8, 128), jnp.int32),
        pltpu.VMEM((_SUBB,), jnp.int32),
        pltpu.VMEM((_SUBB,), jnp.int32),
        pltpu.VMEM((_SUBB,), jnp.int32),
        pltpu.VMEM((_SUBB,), jnp.int32),
        pltpu.VMEM((_SUBB, _D), jnp.float32),
        pltpu.VMEM((_SUBB, _D), jnp.float32),
        pltpu.VMEM((_NW * 16,), jnp.int32),
        pltpu.SemaphoreType.DMA,
        pltpu.SemaphoreType.DMA,
    ],
)
def _segsum(m_hbm, bpk_hbm, cnt_hbm, out_hbm, deg_hbm, acc, dacc, pidx,
            sidx0, sidx1, didx0, didx1, rows0, rows1, cbuf, sem0, sem1):
    w = _wid()
    zero = jnp.zeros((16,), jnp.float32)
    ones = jnp.ones((16,), jnp.float32)
    sidx = (sidx0, sidx1)
    didx = (didx0, didx1)
    rows = (rows0, rows1)
    sems = (sem0, sem1)

    def zbody(r, _):
        for j in range(_D // 16):
            acc[pl.ds(r * _D + 16 * j, 16)] = zero
        dacc[pl.ds(r * 16, 16)] = zero
        return 0

    lax.fori_loop(0, _ROWS + 1, zbody, 0)

    pltpu.sync_copy(
        cnt_hbm.at[pl.ds(pl.multiple_of(w * _NW * 16, 16), _NW * 16)], cbuf)

    def unpack_issue(p, sb):
        for j in range(_SUBB // 16):
            v = pidx[sb, pl.ds(16 * j, 16)]
            sidx[p][pl.ds(16 * j, 16)] = jnp.right_shift(v, 9)
            didx[p][pl.ds(16 * j, 16)] = v & 511
        pltpu.async_copy(m_hbm.at[sidx[p]], rows[p], sems[p])

    def compute(p):
        def group(g, _):
            dvec = didx[p][pl.ds(g * 16, 16)]
            bases = [dvec[i] * _D for i in range(16)]
            for i in range(16):
                plsc.addupdate(dacc.at[pl.ds(dvec[i] * 16, 16)], ones)
            for j in range(_D // 16):
                for i in range(16):
                    plsc.addupdate(
                        acc.at[pl.ds(bases[i] + 16 * j, 16)],
                        rows[p][g * 16 + i, pl.ds(16 * j, 16)])
            return 0

        lax.fori_loop(0, _SUBB // 16, group, 0)

    def seg(tt, _):
        cnt = cbuf[pl.ds(tt * 16, 16)][0]
        nch = (cnt + (_CHUNK - 1)) // _CHUNK
        segrow = (w * _NW + tt) * _CAPR

        def chunk(ch, _):
            ro = pl.multiple_of(segrow + ch * 8, 8)
            pltpu.sync_copy(bpk_hbm.at[pl.ds(ro, 8)], pidx)
            unpack_issue(0, 0)
            for sb in range(_CHUNK // _SUBB):
                p = sb % 2
                if sb + 1 < _CHUNK // _SUBB:
                    unpack_issue(1 - p, sb + 1)
                pltpu.make_async_copy(m_hbm.at[sidx[p]], rows[p],
                                      sems[p]).wait()
                compute(p)
            return 0

        lax.fori_loop(0, nch, chunk, 0)
        return 0

    lax.fori_loop(0, _NW, seg, 0)

    pltpu.sync_copy(acc.at[pl.ds(0, _ROWS * _D)],
                    out_hbm.at[pl.ds(pl.multiple_of(w * _ROWS * _D, 64),
                                     _ROWS * _D)])
    pltpu.sync_copy(dacc.at[pl.ds(0, _ROWS * 16)],
                    deg_hbm.at[pl.ds(pl.multiple_of(w * _ROWS * 16, 16),
                                     _ROWS * 16)])


_R = 1000


def _gelu(x):
    return 0.5 * x * (1.0 + lax.erf(x * 0.7071067811865476))


def _mm(a, b):
    return jnp.dot(a, b, preferred_element_type=jnp.float32)


def _head_body(f_ref, Win_ref, bin_ref, W1_ref, b1_ref, h_ref, m_ref):
    h = _gelu(_mm(f_ref[...], Win_ref[...]) + bin_ref[...])
    h_ref[...] = h
    m_ref[...] = _gelu(_mm(h, W1_ref[...]) + b1_ref[...])


def _mid_body(s_ref, dg_ref, h_ref, W2_ref, b2_ref, W3_ref, b3_ref, Wl_ref,
              bl_ref, W1n_ref, b1n_ref, hn_ref, mn_ref):
    h = h_ref[...]
    inv = 1.0 / jnp.maximum(dg_ref[...][:, :1], 1.0)
    agg = s_ref[...] * inv
    t = _gelu(_mm(agg, W2_ref[...]) + b2_ref[...] + _mm(h, W3_ref[...]) +
              b3_ref[...])
    hn = _mm(t, Wl_ref[...]) + bl_ref[...] + h
    hn_ref[...] = hn
    mn_ref[...] = _gelu(_mm(hn, W1n_ref[...]) + b1n_ref[...])


def _tail_body(s_ref, dg_ref, h_ref, W2_ref, b2_ref, W3_ref, b3_ref, Wl_ref,
               bl_ref, Wo_ref, bo_ref, o_ref):
    h = h_ref[...]
    inv = 1.0 / jnp.maximum(dg_ref[...][:, :1], 1.0)
    agg = s_ref[...] * inv
    t = _gelu(_mm(agg, W2_ref[...]) + b2_ref[...] + _mm(h, W3_ref[...]) +
              b3_ref[...])
    hn = _mm(t, Wl_ref[...]) + bl_ref[...] + h
    o_ref[...] = _mm(hn, Wo_ref[...]) + bo_ref[...]


_rows_spec = pl.BlockSpec((_R, _D), lambda i: (i, 0))
_deg_spec = pl.BlockSpec((_R, 16), lambda i: (i, 0))
_w_spec = pl.BlockSpec((_D, _D), lambda i: (0, 0))
_b_spec = pl.BlockSpec((_D,), lambda i: (0,))
_row_out = jax.ShapeDtypeStruct((_N, _D), jnp.float32)

_head = pl.pallas_call(
    _head_body,
    grid=(_N // _R,),
    in_specs=[_rows_spec, _w_spec, _b_spec, _w_spec, _b_spec],
    out_specs=[_rows_spec, _rows_spec],
    out_shape=[_row_out, _row_out],
)

_mid = pl.pallas_call(
    _mid_body,
    grid=(_N // _R,),
    in_specs=[_rows_spec, _deg_spec, _rows_spec] + [_w_spec, _b_spec] * 4,
    out_specs=[_rows_spec, _rows_spec],
    out_shape=[_row_out, _row_out],
)

_tail = pl.pallas_call(
    _tail_body,
    grid=(_N // _R,),
    in_specs=[_rows_spec, _deg_spec, _rows_spec] + [_w_spec, _b_spec] * 4,
    out_specs=_rows_spec,
    out_shape=_row_out,
)


def kernel(feats, edge_index, W_in, b_in,
           W1_0, b1_0, W2_0, b2_0, W3_0, b3_0, Wl_0, bl_0,
           W1_1, b1_1, W2_1, b2_1, W3_1, b3_1, Wl_1, bl_1,
           W_out, b_out):
    bpk, cnts = _bucketize(edge_index.reshape(-1))
    h, m0 = _head(feats, W_in, b_in, W1_0, b1_0)
    s0, dg = _segsum(m0, bpk, cnts)
    dg = dg.reshape(_NPAD, 16)[:_N]
    h1, m1 = _mid(s0.reshape(_NPAD, _D)[:_N], dg, h, W2_0, b2_0, W3_0, b3_0,
                  Wl_0, bl_0, W1_1, b1_1)
    s1, _ = _segsum(m1, bpk, cnts)
    out = _tail(s1.reshape(_NPAD, _D)[:_N], dg, h1, W2_1, b2_1, W3_1, b3_1,
                Wl_1, bl_1, W_out, b_out)
    return out


# speedup vs baseline: 8.8583x; 8.8583x over previous
import functools

import jax
import jax.numpy as jnp
from jax import lax
from jax.experimental import pallas as pl
from jax.experimental.pallas import tpu as pltpu
from jax.experimental.pallas import tpu_sc as plsc

_N = 10000
_E = 320000
_D = 128

_NW = 32
_ROWS = 313
_NPAD = _NW * _ROWS
_CHUNK = 1024
_SUBB = 128
_EPT = _E // _NW
_CAPR = 80
_BSTRIDE = _CHUNK + 16
_NSEG = _NW * _NW

_mesh = plsc.VectorSubcoreMesh(core_axis_name="c", subcore_axis_name="s")


def _wid():
    return lax.axis_index("s") * 2 + lax.axis_index("c")


def _splat(x):
    return jnp.full((16,), 1, jnp.int32) * x


@functools.partial(
    pl.kernel,
    out_type=[
        jax.ShapeDtypeStruct((_NSEG * _CAPR, 128), jnp.int32),
        jax.ShapeDtypeStruct((_NSEG * 16,), jnp.int32),
    ],
    mesh=_mesh,
    scratch_types=[
        pltpu.VMEM((_EPT + 16,), jnp.int32),
        pltpu.VMEM((_EPT + 16,), jnp.int32),
        pltpu.VMEM((_NW * _BSTRIDE,), jnp.int32),
        pltpu.VMEM((8, 128), jnp.int32),
        pltpu.VMEM((_NW * 16,), jnp.int32),
        pltpu.VMEM((_NW * 16,), jnp.int32),
        pltpu.VMEM((16,), jnp.int32),
    ],
)
def _bucketize(edges_hbm, bpk_hbm, cnt_hbm, sstage, dstage, pbuf, fbuf, fill,
               done, cbuf):
    t = _wid()
    zero16 = jnp.zeros((16,), jnp.int32)

    def zinit(b, _):
        fill[pl.ds(pl.multiple_of(b * 16, 16), 16)] = zero16
        done[pl.ds(pl.multiple_of(b * 16, 16), 16)] = zero16
        return 0

    lax.fori_loop(0, _NW, zinit, 0)

    eo = pl.multiple_of(t * _EPT, 16)
    pltpu.sync_copy(edges_hbm.at[pl.ds(eo, _EPT)], sstage.at[pl.ds(0, _EPT)])
    pltpu.sync_copy(edges_hbm.at[pl.ds(_E + eo, _EPT)],
                    dstage.at[pl.ds(0, _EPT)])

    def _flush_block(b, w0r):
        def mv(r8, _):
            for j in range(8):
                fbuf[r8, pl.ds(16 * j, 16)] = (
                    pbuf[pl.ds(b * _BSTRIDE + r8 * 128 + 16 * j, 16)])
            return 0

        lax.fori_loop(0, 8, mv, 0)
        seg = b * _NW + t
        ro = pl.multiple_of(seg * _CAPR + w0r, 8)
        pltpu.sync_copy(fbuf, bpk_hbm.at[pl.ds(ro, 8)])

    def edge(e, _):
        s = sstage[pl.ds(e, 16)][0]
        d = dstage[pl.ds(e, 16)][0]
        b = d // _ROWS
        loc = d - b * _ROWS
        c = fill[pl.ds(b * 16, 16)][0]
        pbuf[pl.ds(b * _BSTRIDE + c, 16)] = _splat(s * 512 + loc)

        def flush(_):
            w0 = done[pl.ds(b * 16, 16)][0]
            _flush_block(b, w0)
            done[pl.ds(b * 16, 16)] = _splat(w0 + 8)
            fill[pl.ds(b * 16, 16)] = zero16
            return 0

        def keep(_):
            fill[pl.ds(b * 16, 16)] = _splat(c + 1)
            return 0

        lax.cond(c + 1 >= _CHUNK, flush, keep, 0)
        return 0

    lax.fori_loop(0, _EPT, edge, 0)

    pad_d = jnp.full((16,), _ROWS, jnp.int32)

    def fin(b, _):
        c = fill[pl.ds(b * 16, 16)][0]
        for j in range(64):
            @pl.when(c + 16 * j < _CHUNK)
            def _():
                pbuf[pl.ds(b * _BSTRIDE + c + 16 * j, 16)] = pad_d
        w0 = done[pl.ds(b * 16, 16)][0]
        _flush_block(b, w0)
        cbuf[pl.ds(0, 16)] = _splat(w0 * 128 + c)
        co = pl.multiple_of((b * _NW + t) * 16, 16)
        pltpu.sync_copy(cbuf, cnt_hbm.at[pl.ds(co, 16)])
        return 0

    lax.fori_loop(0, _NW, fin, 0)


@functools.partial(
    pl.kernel,
    out_type=[
        jax.ShapeDtypeStruct((_NPAD * _D,), jnp.float32),
        jax.ShapeDtypeStruct((_NPAD * 16,), jnp.float32),
    ],
    mesh=_mesh,
    scratch_types=[
        pltpu.VMEM(((_ROWS + 1) * _D,), jnp.float32),
        pltpu.VMEM(((_ROWS + 1) * 16,), jnp.float32),
        pltpu.VMEM((8, 128), jnp.int32),
        pltpu.VMEM((_SUBB,), jnp.int32),
        pltpu.VMEM((_SUBB,), jnp.int32),
        pltpu.VMEM((_SUBB,), jnp.int32),
        pltpu.VMEM((_SUBB,), jnp.int32),
        pltpu.VMEM((_SUBB, _D), jnp.float32),
        pltpu.VMEM((_SUBB, _D), jnp.float32),
        pltpu.VMEM((_NW * 16,), jnp.int32),
        pltpu.SemaphoreType.DMA,
        pltpu.SemaphoreType.DMA,
    ],
)
def _segsum(m_hbm, bpk_hbm, cnt_hbm, out_hbm, deg_hbm, acc, dacc, pidx,
            sidx0, sidx1, didx0, didx1, rows0, rows1, cbuf, sem0, sem1):
    w = _wid()
    zero = jnp.zeros((16,), jnp.float32)
    ones = jnp.ones((16,), jnp.float32)
    sidx = (sidx0, sidx1)
    didx = (didx0, didx1)
    rows = (rows0, rows1)
    sems = (sem0, sem1)

    def zbody(r, _):
        for j in range(_D // 16):
            acc[pl.ds(r * _D + 16 * j, 16)] = zero
        dacc[pl.ds(r * 16, 16)] = zero
        return 0

    lax.fori_loop(0, _ROWS + 1, zbody, 0)

    pltpu.sync_copy(
        cnt_hbm.at[pl.ds(pl.multiple_of(w * _NW * 16, 16), _NW * 16)], cbuf)

    def unpack_issue(p, sb):
        for j in range(_SUBB // 16):
            v = pidx[sb, pl.ds(16 * j, 16)]
            sidx[p][pl.ds(16 * j, 16)] = jnp.right_shift(v, 9)
            didx[p][pl.ds(16 * j, 16)] = v & 511
        pltpu.async_copy(m_hbm.at[sidx[p]], rows[p], sems[p])

    def compute(p):
        def group(g, _):
            dvec = didx[p][pl.ds(g * 16, 16)]
            bases = [dvec[i] * _D for i in range(16)]
            for i in range(16):
                plsc.addupdate(dacc.at[pl.ds(dvec[i] * 16, 16)], ones)
            for j in range(_D // 16):
                for i in range(16):
                    plsc.addupdate(
                        acc.at[pl.ds(bases[i] + 16 * j, 16)],
                        rows[p][g * 16 + i, pl.ds(16 * j, 16)])
            return 0

        lax.fori_loop(0, _SUBB // 16, group, 0)

    def seg(tt, _):
        cnt = cbuf[pl.ds(tt * 16, 16)][0]
        nch = (cnt + (_CHUNK - 1)) // _CHUNK
        segrow = (w * _NW + tt) * _CAPR

        def chunk(ch, _):
            ro = pl.multiple_of(segrow + ch * 8, 8)
            pltpu.sync_copy(bpk_hbm.at[pl.ds(ro, 8)], pidx)
            left = cnt - ch * _CHUNK
            nsb = (jnp.minimum(left, _CHUNK) + (_SUBB - 1)) // _SUBB

            def sub(sb, _):
                for j in range(_SUBB // 16):
                    v = pidx[sb, pl.ds(16 * j, 16)]
                    sidx[0][pl.ds(16 * j, 16)] = jnp.right_shift(v, 9)
                    didx[0][pl.ds(16 * j, 16)] = v & 511
                pltpu.async_copy(m_hbm.at[sidx[0]], rows[0], sems[0]).wait()
                compute(0)
                return 0

            lax.fori_loop(0, nsb, sub, 0)
            return 0

        lax.fori_loop(0, nch, chunk, 0)
        return 0

    lax.fori_loop(0, _NW, seg, 0)

    pltpu.sync_copy(acc.at[pl.ds(0, _ROWS * _D)],
                    out_hbm.at[pl.ds(pl.multiple_of(w * _ROWS * _D, 64),
                                     _ROWS * _D)])
    pltpu.sync_copy(dacc.at[pl.ds(0, _ROWS * 16)],
                    deg_hbm.at[pl.ds(pl.multiple_of(w * _ROWS * 16, 16),
                                     _ROWS * 16)])


_R = 1000


def _gelu(x):
    return 0.5 * x * (1.0 + lax.erf(x * 0.7071067811865476))


def _mm(a, b):
    return jnp.dot(a, b, preferred_element_type=jnp.float32)


def _head_body(f_ref, Win_ref, bin_ref, W1_ref, b1_ref, h_ref, m_ref):
    h = _gelu(_mm(f_ref[...], Win_ref[...]) + bin_ref[...])
    h_ref[...] = h
    m_ref[...] = _gelu(_mm(h, W1_ref[...]) + b1_ref[...])


def _mid_body(s_ref, dg_ref, h_ref, W2_ref, b2_ref, W3_ref, b3_ref, Wl_ref,
              bl_ref, W1n_ref, b1n_ref, hn_ref, mn_ref):
    h = h_ref[...]
    inv = 1.0 / jnp.maximum(dg_ref[...][:, :1], 1.0)
    agg = s_ref[...] * inv
    t = _gelu(_mm(agg, W2_ref[...]) + b2_ref[...] + _mm(h, W3_ref[...]) +
              b3_ref[...])
    hn = _mm(t, Wl_ref[...]) + bl_ref[...] + h
    hn_ref[...] = hn
    mn_ref[...] = _gelu(_mm(hn, W1n_ref[...]) + b1n_ref[...])


def _tail_body(s_ref, dg_ref, h_ref, W2_ref, b2_ref, W3_ref, b3_ref, Wl_ref,
               bl_ref, Wo_ref, bo_ref, o_ref):
    h = h_ref[...]
    inv = 1.0 / jnp.maximum(dg_ref[...][:, :1], 1.0)
    agg = s_ref[...] * inv
    t = _gelu(_mm(agg, W2_ref[...]) + b2_ref[...] + _mm(h, W3_ref[...]) +
              b3_ref[...])
    hn = _mm(t, Wl_ref[...]) + bl_ref[...] + h
    o_ref[...] = _mm(hn, Wo_ref[...]) + bo_ref[...]


_rows_spec = pl.BlockSpec((_R, _D), lambda i: (i, 0))
_deg_spec = pl.BlockSpec((_R, 16), lambda i: (i, 0))
_w_spec = pl.BlockSpec((_D, _D), lambda i: (0, 0))
_b_spec = pl.BlockSpec((_D,), lambda i: (0,))
_row_out = jax.ShapeDtypeStruct((_N, _D), jnp.float32)

_head = pl.pallas_call(
    _head_body,
    grid=(_N // _R,),
    in_specs=[_rows_spec, _w_spec, _b_spec, _w_spec, _b_spec],
    out_specs=[_rows_spec, _rows_spec],
    out_shape=[_row_out, _row_out],
)

_mid = pl.pallas_call(
    _mid_body,
    grid=(_N // _R,),
    in_specs=[_rows_spec, _deg_spec, _rows_spec] + [_w_spec, _b_spec] * 4,
    out_specs=[_rows_spec, _rows_spec],
    out_shape=[_row_out, _row_out],
)

_tail = pl.pallas_call(
    _tail_body,
    grid=(_N // _R,),
    in_specs=[_rows_spec, _deg_spec, _rows_spec] + [_w_spec, _b_spec] * 4,
    out_specs=_rows_spec,
    out_shape=_row_out,
)


def kernel(feats, edge_index, W_in, b_in,
           W1_0, b1_0, W2_0, b2_0, W3_0, b3_0, Wl_0, bl_0,
           W1_1, b1_1, W2_1, b2_1, W3_1, b3_1, Wl_1, bl_1,
           W_out, b_out):
    bpk, cnts = _bucketize(edge_index.reshape(-1))
    h, m0 = _head(feats, W_in, b_in, W1_0, b1_0)
    s0, dg = _segsum(m0, bpk, cnts)
    dg = dg.reshape(_NPAD, 16)[:_N]
    h1, m1 = _mid(s0.reshape(_NPAD, _D)[:_N], dg, h, W2_0, b2_0, W3_0, b3_0,
                  Wl_0, bl_0, W1_1, b1_1)
    s1, _ = _segsum(m1, bpk, cnts)
    out = _tail(s1.reshape(_NPAD, _D)[:_N], dg, h1, W2_1, b2_1, W3_1, b3_1,
                Wl_1, bl_1, W_out, b_out)
    return out
